# baseline (speedup 1.0000x reference)
import functools

import numpy as np
import jax
import jax.numpy as jnp
from jax import lax
from jax.experimental import pallas as pl
from jax.experimental.pallas import tpu as pltpu
from jax.experimental.pallas import tpu_sc as plsc

H = 28
W = 28
PIX = H * W
B = 2048
NPTS = 128
NSEG = NPTS - 1
NW = 32
SPT = B // NW
NGRP = SPT // 16
SROW = PIX + 1
CANVAS_WORDS = SPT * SROW

_EPS = np.float32(1e-7)
_P1 = np.float32(1.0) - _EPS
L0 = float(np.float32(np.log(np.float64(_EPS))))
L1 = float(np.float32(np.log(np.float64(_P1))))
L2 = float(np.float32(np.log(np.float64(np.float32(1.0) - _P1))))

_mesh = plsc.VectorSubcoreMesh(core_axis_name="c", subcore_axis_name="s")


@functools.partial(
    pl.kernel,
    out_type=jax.ShapeDtypeStruct((NW, 16), jnp.float32),
    mesh=_mesh,
    compiler_params=pltpu.CompilerParams(needs_layout_passes=False),
    scratch_types=[
        pltpu.VMEM((SPT * NPTS * 2,), jnp.float32),
        pltpu.VMEM((NPTS * 2 * 65,), jnp.float32),
        pltpu.VMEM((CANVAS_WORDS,), jnp.float32),
        pltpu.VMEM((2, 16 * PIX), jnp.float32),
        pltpu.VMEM((16,), jnp.float32),
        pltpu.SemaphoreType.DMA,
        pltpu.SemaphoreType.DMA,
    ],
)
def _sc_raster_bce(coords_hbm, bitmap_hbm, out_hbm, coords_v, coords_t,
                   canvas_v, tgt_v, acc_v, sem0, sem1):
    wid = lax.axis_index("s") * 2 + lax.axis_index("c")
    sems = [sem0, sem1]

    cps = [
        pltpu.async_copy(bitmap_hbm.at[wid, c], tgt_v.at[c], sems[c])
        for c in range(2)
    ]
    pltpu.sync_copy(coords_hbm.at[wid], coords_v)

    zeros16 = jnp.zeros((16,), jnp.float32)
    ones16 = jnp.ones((16,), jnp.float32)
    iota16 = lax.iota(jnp.int32, 16)

    def zero_body(k, carry):
        for u in range(16):
            canvas_v[pl.ds(k * 256 + u * 16, 16)] = zeros16
        return carry

    lax.fori_loop(0, CANVAS_WORDS // 256, zero_body, 0)
    for r in range(CANVAS_WORDS // 256 * 256, CANVAS_WORDS, 16):
        canvas_v[pl.ds(r, 16)] = zeros16

    iota65 = lax.iota(jnp.int32, 16) * 65

    def tr_body(j, carry):
        for c in range(16):
            src = coords_v[pl.ds(j * 256 + c * 16, 16)]
            plsc.store_scatter(coords_t, [iota65 + (c * 16 * 65 + j)], src)
        return carry

    lax.fori_loop(0, SPT, tr_body, 0)

    def seg_body(s, carry):
        sb = s * 130

        def grp_body(g, carry2):
            g16 = g * 16
            x0 = (coords_t[pl.ds(sb + g16, 16)] * 27.0).astype(jnp.int32)
            y0 = (coords_t[pl.ds(sb + 65 + g16, 16)] * 27.0).astype(jnp.int32)
            x1 = (coords_t[pl.ds(sb + 130 + g16, 16)] * 27.0).astype(jnp.int32)
            y1 = (coords_t[pl.ds(sb + 195 + g16, 16)] * 27.0).astype(jnp.int32)
            steep = jnp.abs(y1 - y0) > jnp.abs(x1 - x0)
            ma = jnp.where(steep, y0, x0)
            na = jnp.where(steep, x0, y0)
            mb = jnp.where(steep, y1, x1)
            nb = jnp.where(steep, x1, y1)
            sw = ma > mb
            m0 = jnp.where(sw, mb, ma)
            m1 = jnp.where(sw, ma, mb)
            n0 = jnp.where(sw, nb, na)
            n1 = jnp.where(sw, na, nb)
            d = m1 - m0
            ystep = jnp.where(n0 < n1, 1, -1)
            dq = jnp.abs(n1 - n0).astype(jnp.float32) / jnp.maximum(
                d, 1).astype(jnp.float32)
            stride = jnp.where(steep, W, 1)
            nmul = jnp.where(steep, 1, W)
            nys = nmul * ystep
            base = (g * 16 + iota16) * SROW + stride * m0 + nmul * n0

            def step_body(u, carry3):
                for v in range(4):
                    i = u * 4 + v
                    covered = i <= d
                    q = (dq * i.astype(jnp.float32)
                         + 0.001953125).astype(jnp.int32)
                    idx = base + stride * i + nys * q
                    plsc.store_scatter(canvas_v, [idx], ones16, mask=covered)
                return carry3

            lax.fori_loop(0, W // 4, step_body, 0)
            return carry2

        lax.fori_loop(0, NGRP, grp_body, 0)
        return carry

    lax.fori_loop(0, NSEG, seg_body, 0)

    a_set = jnp.full((16,), -L2, jnp.float32)
    a_unset = jnp.full((16,), -L1, jnp.float32)
    b_set = jnp.full((16,), L2 - L1, jnp.float32)
    b_unset = jnp.full((16,), L1 - L0, jnp.float32)

    gacc = zeros16
    for c in range(NGRP):
        buf = c % 2
        cps[buf].wait()

        def smp_body(j, acc, _c=c, _buf=buf):
            def pix_body(k, sacc, _u=7):
                for u in range(_u):
                    kk = k * _u + u
                    cv = canvas_v[pl.ds((_c * 16 + j) * SROW + kk * 16, 16)]
                    t = tgt_v[_buf, pl.ds(j * PIX + kk * 16, 16)]
                    m = cv > 0.5
                    sacc = sacc + jnp.where(m, a_set, a_unset) + jnp.where(
                        m, b_set, b_unset) * t
                return sacc
            sacc = lax.fori_loop(0, PIX // 16 // 7, pix_body, zeros16)
            return acc + sacc

        gacc = lax.fori_loop(0, 16, smp_body, gacc)
        if c + 2 < NGRP:
            cps[buf] = pltpu.async_copy(
                bitmap_hbm.at[wid, c + 2], tgt_v.at[buf], sems[buf])

    acc_v[...] = gacc
    pltpu.sync_copy(acc_v, out_hbm.at[wid])


def _tc_combine_body(p_ref, t_ref, part_ref, c_ref, r_ref, tot_ref):
    diff = p_ref[...] - t_ref[...]
    coord = jnp.sum(diff * diff) * np.float32(1.0 / (B * NPTS * 2))
    raster = jnp.sum(part_ref[...]) * np.float32(1.0 / (B * PIX))
    c_ref[0, 0] = coord
    r_ref[0, 0] = raster
    tot_ref[0, 0] = coord + 0.5 * raster


_tc_combine = pl.pallas_call(
    _tc_combine_body,
    out_shape=[jax.ShapeDtypeStruct((1, 1), jnp.float32)] * 3,
    out_specs=[pl.BlockSpec(memory_space=pltpu.SMEM)] * 3,
)


def kernel(pred_coords, target_coords, target_bitmap):
    coords_slab = pred_coords.reshape(NW, SPT * NPTS * 2)
    bitmap_slab = target_bitmap.reshape(NW, NGRP, 16 * PIX)
    partials = _sc_raster_bce(coords_slab, bitmap_slab)
    p2 = pred_coords.reshape(B, NPTS * 2)
    t2 = target_coords.reshape(B, NPTS * 2)
    coord, raster, total = _tc_combine(p2, t2, partials)
    return (coord[0, 0], raster[0, 0], total[0, 0])


# speedup vs baseline: 2.9999x; 2.9999x over previous
import functools

import numpy as np
import jax
import jax.numpy as jnp
from jax import lax
from jax.experimental import pallas as pl
from jax.experimental.pallas import tpu as pltpu
from jax.experimental.pallas import tpu_sc as plsc

H = 28
W = 28
PIX = H * W
B = 2048
NPTS = 128
NSEG = NPTS - 1
NW = 32
SPT = B // NW
NGRP = SPT // 16
SROW = PIX + 1
CANVAS_WORDS = SPT * SROW

_EPS = np.float32(1e-7)
_P1 = np.float32(1.0) - _EPS
L0 = float(np.float32(np.log(np.float64(_EPS))))
L1 = float(np.float32(np.log(np.float64(_P1))))
L2 = float(np.float32(np.log(np.float64(np.float32(1.0) - _P1))))

_mesh = plsc.VectorSubcoreMesh(core_axis_name="c", subcore_axis_name="s")


@functools.partial(
    pl.kernel,
    out_type=jax.ShapeDtypeStruct((NW, 16), jnp.float32),
    mesh=_mesh,
    compiler_params=pltpu.CompilerParams(needs_layout_passes=False),
    scratch_types=[
        pltpu.VMEM((NPTS, 2, SPT), jnp.float32),
        pltpu.VMEM((CANVAS_WORDS,), jnp.float32),
        pltpu.VMEM((2, 16 * PIX), jnp.float32),
        pltpu.VMEM((16,), jnp.float32),
        pltpu.SemaphoreType.DMA,
        pltpu.SemaphoreType.DMA,
    ],
)
def _sc_raster_bce(coords_hbm, bitmap_hbm, out_hbm, coords_v, canvas_v,
                   tgt_v, acc_v, sem0, sem1):
    wid = lax.axis_index("s") * 2 + lax.axis_index("c")
    sems = [sem0, sem1]

    cps = [
        pltpu.async_copy(bitmap_hbm.at[wid, c], tgt_v.at[c], sems[c])
        for c in range(2)
    ]
    pltpu.sync_copy(coords_hbm.at[wid], coords_v)

    zeros16 = jnp.zeros((16,), jnp.float32)
    ones16 = jnp.ones((16,), jnp.float32)
    iota16 = lax.iota(jnp.int32, 16)

    def zero_body(k, carry):
        for u in range(16):
            canvas_v[pl.ds(k * 256 + u * 16, 16)] = zeros16
        return carry

    lax.fori_loop(0, CANVAS_WORDS // 256, zero_body, 0)
    for r in range(CANVAS_WORDS // 256 * 256, CANVAS_WORDS, 16):
        canvas_v[pl.ds(r, 16)] = zeros16

    def seg_body(s, carry):
        def grp_body(g, carry2):
            sl = pl.ds(g * 16, 16)
            x0 = (coords_v[s, 0, sl] * 27.0).astype(jnp.int32)
            y0 = (coords_v[s, 1, sl] * 27.0).astype(jnp.int32)
            x1 = (coords_v[s + 1, 0, sl] * 27.0).astype(jnp.int32)
            y1 = (coords_v[s + 1, 1, sl] * 27.0).astype(jnp.int32)
            steep = jnp.abs(y1 - y0) > jnp.abs(x1 - x0)
            ma = jnp.where(steep, y0, x0)
            na = jnp.where(steep, x0, y0)
            mb = jnp.where(steep, y1, x1)
            nb = jnp.where(steep, x1, y1)
            sw = ma > mb
            m0 = jnp.where(sw, mb, ma)
            m1 = jnp.where(sw, ma, mb)
            n0 = jnp.where(sw, nb, na)
            n1 = jnp.where(sw, na, nb)
            d = m1 - m0
            ystep = jnp.where(n0 < n1, 1, -1)
            dq = jnp.abs(n1 - n0).astype(jnp.float32) / jnp.maximum(
                d, 1).astype(jnp.float32)
            stride = jnp.where(steep, W, 1)
            nmul = jnp.where(steep, 1, W)
            nys = nmul * ystep
            base = (g * 16 + iota16) * SROW + stride * m0 + nmul * n0

            def step_body(u, carry3):
                for v in range(4):
                    i = u * 4 + v
                    covered = i <= d
                    q = (dq * i.astype(jnp.float32)
                         + 0.001953125).astype(jnp.int32)
                    idx = base + stride * i + nys * q
                    plsc.store_scatter(canvas_v, [idx], ones16, mask=covered)
                return carry3

            lax.fori_loop(0, W // 4, step_body, 0)
            return carry2

        lax.fori_loop(0, NGRP, grp_body, 0)
        return carry

    lax.fori_loop(0, NSEG, seg_body, 0)

    a_set = jnp.full((16,), -L2, jnp.float32)
    a_unset = jnp.full((16,), -L1, jnp.float32)
    b_set = jnp.full((16,), L2 - L1, jnp.float32)
    b_unset = jnp.full((16,), L1 - L0, jnp.float32)

    gacc = zeros16
    for c in range(NGRP):
        buf = c % 2
        cps[buf].wait()

        def smp_body(j, acc, _c=c, _buf=buf):
            def pix_body(k, sacc, _u=7):
                for u in range(_u):
                    kk = k * _u + u
                    cv = canvas_v[pl.ds((_c * 16 + j) * SROW + kk * 16, 16)]
                    t = tgt_v[_buf, pl.ds(j * PIX + kk * 16, 16)]
                    m = cv > 0.5
                    sacc = sacc + jnp.where(m, a_set, a_unset) + jnp.where(
                        m, b_set, b_unset) * t
                return sacc
            sacc = lax.fori_loop(0, PIX // 16 // 7, pix_body, zeros16)
            return acc + sacc

        gacc = lax.fori_loop(0, 16, smp_body, gacc)
        if c + 2 < NGRP:
            cps[buf] = pltpu.async_copy(
                bitmap_hbm.at[wid, c + 2], tgt_v.at[buf], sems[buf])

    acc_v[...] = gacc
    pltpu.sync_copy(acc_v, out_hbm.at[wid])


def _tc_combine_body(p_ref, t_ref, part_ref, c_ref, r_ref, tot_ref):
    diff = p_ref[...] - t_ref[...]
    coord = jnp.sum(diff * diff) * np.float32(1.0 / (B * NPTS * 2))
    raster = jnp.sum(part_ref[...]) * np.float32(1.0 / (B * PIX))
    c_ref[0, 0] = coord
    r_ref[0, 0] = raster
    tot_ref[0, 0] = coord + 0.5 * raster


_tc_combine = pl.pallas_call(
    _tc_combine_body,
    out_shape=[jax.ShapeDtypeStruct((1, 1), jnp.float32)] * 3,
    out_specs=[pl.BlockSpec(memory_space=pltpu.SMEM)] * 3,
)


def kernel(pred_coords, target_coords, target_bitmap):
    coords_slab = pred_coords.reshape(NW, SPT, NPTS, 2).transpose(0, 2, 3, 1)
    bitmap_slab = target_bitmap.reshape(NW, NGRP, 16 * PIX)
    partials = _sc_raster_bce(coords_slab, bitmap_slab)
    p2 = pred_coords.reshape(B, NPTS * 2)
    t2 = target_coords.reshape(B, NPTS * 2)
    coord, raster, total = _tc_combine(p2, t2, partials)
    return (coord[0, 0], raster[0, 0], total[0, 0])
